# baseline (speedup 1.0000x reference)
import functools

import jax
import jax.numpy as jnp
from jax import lax
from jax.experimental import pallas as pl
from jax.experimental.pallas import tpu as pltpu
from jax.experimental.pallas import tpu_sc as plsc

N = 10000
E = 320000
D = 128
NC = 2
NS = 16
NW = NC * NS
EPT = E // NW
BLK = 100
NB = EPT // BLK
NQ = 10
NBQ = NB // NQ
RPT = N // NS
WR = 624
WRL = N - 15 * WR


def _agg_body(x_hbm, ei_hbm, out_hbm,
              isrc0, isrc1, idst0, idst1, rows0, rows1, rows2, acc,
              gs0, gs1, gs2, ss0, ss1, ss2, isem):
    c = lax.axis_index("c")
    s = lax.axis_index("s")
    wid = s * NC + c
    bufs = (rows0, rows1, rows2)
    gsems = (gs0, gs1, gs2)
    ssems = (ss0, ss1, ss2)
    isrc = (isrc0, isrc1)
    idst = (idst0, idst1)

    @pl.when(jnp.logical_and(c == 0, s < NS - 1))
    def _():
        pltpu.sync_copy(x_hbm.at[pl.ds(s * WR, WR)],
                        acc.at[pl.ds(s * WR, WR)])

    @pl.when(jnp.logical_and(c == 0, s == NS - 1))
    def _():
        pltpu.sync_copy(x_hbm.at[pl.ds((NS - 1) * WR, WRL)],
                        acc.at[pl.ds((NS - 1) * WR, WRL)])

    @pl.when(c == 1)
    def _():
        zv = jnp.zeros((16,), jnp.float32)

        def zrow(i, carry):
            for j2 in range(D // 16):
                rows0[i, pl.ds(j2 * 16, 16)] = zv
            return carry

        lax.fori_loop(0, BLK, zrow, 0)
        for j in range(RPT // BLK):
            pltpu.sync_copy(rows0, acc.at[pl.ds(s * RPT + j * BLK, BLK)])
        pltpu.sync_copy(rows0.at[pl.ds(0, RPT % BLK)],
                        acc.at[pl.ds(s * RPT + (RPT // BLK) * BLK, RPT % BLK)])

    plsc.subcore_barrier()

    def start_gather(q, b, m):
        pltpu.async_copy(x_hbm.at[isrc[q % 2].at[b]], bufs[m], gsems[m])

    def wait_gather(q, b, m):
        pltpu.make_async_copy(x_hbm.at[isrc[q % 2].at[b]], bufs[m],
                              gsems[m]).wait()

    def stage_idx(q):
        return (
            pltpu.async_copy(ei_hbm.at[0, wid, q], isrc[q % 2], isem),
            pltpu.async_copy(ei_hbm.at[1, wid, q], idst[q % 2], isem),
        )

    def start_scatter(q, b, m):
        pltpu.async_copy(bufs[m], acc.at[idst[q % 2].at[b]], ssems[m],
                         add=True)

    def wait_scatter(q, b, m):
        pltpu.make_async_copy(bufs[m], acc.at[idst[q % 2].at[b]],
                              ssems[m]).wait()

    def block(q, b, m, prev=True, prev_qb=None, pf=None):
        wait_gather(q, b, m)
        start_scatter(q, b, m)
        if prev:
            pq, pb = prev_qb if prev_qb is not None else (q, b - 1)
            wait_scatter(pq, pb, (m - 1) % 3)
        if pf is not None:
            start_gather(pf[0], pf[1], (m + 2) % 3)

    d0, d1 = stage_idx(0)
    d0.wait()
    d1.wait()
    start_gather(0, 0, 0)
    start_gather(0, 1, 1)

    for q in range(NQ):
        mq = (q * NBQ) % 3
        block(q, 0, mq, prev=(q > 0), prev_qb=(q - 1, NBQ - 1), pf=(q, 2))
        stage = stage_idx(q + 1) if q < NQ - 1 else None

        def body(k, carry, q=q, mq=mq):
            b = 3 * k + 1
            block(q, b, (mq + 1) % 3, pf=(q, b + 2))
            block(q, b + 1, (mq + 2) % 3, pf=(q, b + 3))
            block(q, b + 2, mq, pf=(q, b + 4))
            return carry

        lax.fori_loop(0, (NBQ - 4) // 3, body, 0)
        block(q, NBQ - 3, (mq + NBQ - 3) % 3, pf=(q, NBQ - 1))
        if stage is not None:
            stage[0].wait()
            stage[1].wait()
            block(q, NBQ - 2, (mq + NBQ - 2) % 3, pf=(q + 1, 0))
            block(q, NBQ - 1, (mq + NBQ - 1) % 3, pf=(q + 1, 1))
        else:
            block(q, NBQ - 2, (mq + NBQ - 2) % 3)
            block(q, NBQ - 1, (mq + NBQ - 1) % 3)
    wait_scatter(NQ - 1, NBQ - 1, (NQ * NBQ - 1) % 3)

    plsc.subcore_barrier()
    @pl.when(s < NS - 1)
    def _():
        pltpu.sync_copy(acc.at[pl.ds(s * WR, WR)],
                        out_hbm.at[c, pl.ds(s * WR, WR)])

    @pl.when(s == NS - 1)
    def _():
        pltpu.sync_copy(acc.at[pl.ds((NS - 1) * WR, WRL)],
                        out_hbm.at[c, pl.ds((NS - 1) * WR, WRL)])


def _make_agg():
    mesh = plsc.VectorSubcoreMesh(core_axis_name="c", subcore_axis_name="s",
                                  num_cores=NC, num_subcores=NS)
    return functools.partial(
        pl.kernel,
        out_type=jax.ShapeDtypeStruct((NC, N, D), jnp.float32),
        mesh=mesh,
        scratch_types=[
            pltpu.VMEM((NBQ, BLK), jnp.int32),
            pltpu.VMEM((NBQ, BLK), jnp.int32),
            pltpu.VMEM((NBQ, BLK), jnp.int32),
            pltpu.VMEM((NBQ, BLK), jnp.int32),
            pltpu.VMEM((BLK, D), jnp.float32),
            pltpu.VMEM((BLK, D), jnp.float32),
            pltpu.VMEM((BLK, D), jnp.float32),
            pltpu.VMEM_SHARED((N, D), jnp.float32),
            pltpu.SemaphoreType.DMA,
            pltpu.SemaphoreType.DMA,
            pltpu.SemaphoreType.DMA,
            pltpu.SemaphoreType.DMA,
            pltpu.SemaphoreType.DMA,
            pltpu.SemaphoreType.DMA,
            pltpu.SemaphoreType.DMA,
        ],
    )(_agg_body)


_agg_cache = []


def _get_agg():
    if not _agg_cache:
        _agg_cache.append(_make_agg())
    return _agg_cache[0]


def _mlp1_body(agg_ref, wa_ref, ba_ref, wb_ref, bb_ref, o_ref):
    h = agg_ref[0] + agg_ref[1]
    t = jnp.dot(h, wa_ref[...], preferred_element_type=jnp.float32) + ba_ref[...]
    t = jnp.maximum(t, 0.0)
    u = jnp.dot(t, wb_ref[...], preferred_element_type=jnp.float32) + bb_ref[...]
    o_ref[...] = jnp.maximum(u, 0.0)


def _mlp2_body(agg_ref, wa_ref, ba_ref, wb_ref, bb_ref,
               wf_ref, bf_ref, o_ref):
    h = agg_ref[0] + agg_ref[1]
    t = jnp.dot(h, wa_ref[...], preferred_element_type=jnp.float32) + ba_ref[...]
    t = jnp.maximum(t, 0.0)
    u = jnp.dot(t, wb_ref[...], preferred_element_type=jnp.float32) + bb_ref[...]
    o_ref[...] = jnp.dot(u, wf_ref[...], preferred_element_type=jnp.float32) + bf_ref[...]


BN = 2000


def _w_spec():
    return pl.BlockSpec((D, D), lambda i: (0, 0))


def _b_spec():
    return pl.BlockSpec((D,), lambda i: (0,))


def _make_mlp1():
    return pl.pallas_call(
        _mlp1_body,
        grid=(N // BN,),
        in_specs=[
            pl.BlockSpec((NC, BN, D), lambda i: (0, i, 0)),
            _w_spec(), _b_spec(), _w_spec(), _b_spec(),
        ],
        out_specs=pl.BlockSpec((BN, D), lambda i: (i, 0)),
        out_shape=jax.ShapeDtypeStruct((N, D), jnp.float32),
    )


def _make_mlp2():
    return pl.pallas_call(
        _mlp2_body,
        grid=(N // BN,),
        in_specs=[
            pl.BlockSpec((NC, BN, D), lambda i: (0, i, 0)),
            _w_spec(), _b_spec(), _w_spec(), _b_spec(), _w_spec(), _b_spec(),
        ],
        out_specs=pl.BlockSpec((BN, D), lambda i: (i, 0)),
        out_shape=jax.ShapeDtypeStruct((N, D), jnp.float32),
    )


_mlp1 = _make_mlp1()
_mlp2 = _make_mlp2()


@jax.jit
def kernel(x, edge_index, W1a, b1a, W1b, b1b, W2a, b2a, W2b, b2b, Wfc, bfc):
    ei = edge_index.astype(jnp.int32).reshape(2, NW, NQ, NBQ, BLK)

    agg = _get_agg()
    agg1 = agg(x, ei)
    h1 = _mlp1(agg1, W1a, b1a, W1b, b1b)
    agg2 = agg(h1, ei)
    out = _mlp2(agg2, W2a, b2a, W2b, b2b, Wfc, bfc)
    return out


# speedup vs baseline: 1.0200x; 1.0200x over previous
import functools

import jax
import jax.numpy as jnp
from jax import lax
from jax.experimental import pallas as pl
from jax.experimental.pallas import tpu as pltpu
from jax.experimental.pallas import tpu_sc as plsc

N = 10000
E = 320000
D = 128
NC = 2
NS = 16
NW = NC * NS
EPT = E // NW
BLK = 100
NB = EPT // BLK
NQ = 10
NBQ = NB // NQ
RPT = N // NS
WR = 624
WRL = N - 15 * WR


def _agg_body(x_hbm, ei_hbm, out_hbm,
              isrc0, isrc1, idst0, idst1, rows0, rows1, rows2, acc,
              gs0, gs1, gs2, ss0, ss1, ss2, isem):
    c = lax.axis_index("c")
    s = lax.axis_index("s")
    wid = s * NC + c
    bufs = (rows0, rows1, rows2)
    gsems = (gs0, gs1, gs2)
    ssems = (ss0, ss1, ss2)
    isrc = (isrc0, isrc1)
    idst = (idst0, idst1)

    zv = jnp.zeros((16,), jnp.float32)

    def zrow(i, carry):
        for j2 in range(D // 16):
            rows0[i, pl.ds(j2 * 16, 16)] = zv
        return carry

    lax.fori_loop(0, BLK, zrow, 0)
    for j in range(RPT // BLK):
        pltpu.sync_copy(rows0, acc.at[pl.ds(s * RPT + j * BLK, BLK)])
    pltpu.sync_copy(rows0.at[pl.ds(0, RPT % BLK)],
                    acc.at[pl.ds(s * RPT + (RPT // BLK) * BLK, RPT % BLK)])
    plsc.subcore_barrier()

    def start_gather(q, b, m):
        pltpu.async_copy(x_hbm.at[isrc[q % 2].at[b]], bufs[m], gsems[m])

    def wait_gather(q, b, m):
        pltpu.make_async_copy(x_hbm.at[isrc[q % 2].at[b]], bufs[m],
                              gsems[m]).wait()

    def stage_idx(q):
        return (
            pltpu.async_copy(ei_hbm.at[0, wid, q], isrc[q % 2], isem),
            pltpu.async_copy(ei_hbm.at[1, wid, q], idst[q % 2], isem),
        )

    def start_scatter(q, b, m):
        pltpu.async_copy(bufs[m], acc.at[idst[q % 2].at[b]], ssems[m],
                         add=True)

    def wait_scatter(q, b, m):
        pltpu.make_async_copy(bufs[m], acc.at[idst[q % 2].at[b]],
                              ssems[m]).wait()

    def block(q, b, m, prev=True, prev_qb=None, pf=None):
        wait_gather(q, b, m)
        start_scatter(q, b, m)
        if prev:
            pq, pb = prev_qb if prev_qb is not None else (q, b - 1)
            wait_scatter(pq, pb, (m - 1) % 3)
        if pf is not None:
            start_gather(pf[0], pf[1], (m + 2) % 3)

    d0, d1 = stage_idx(0)
    d0.wait()
    d1.wait()
    start_gather(0, 0, 0)
    start_gather(0, 1, 1)

    for q in range(NQ):
        mq = (q * NBQ) % 3
        block(q, 0, mq, prev=(q > 0), prev_qb=(q - 1, NBQ - 1), pf=(q, 2))
        stage = stage_idx(q + 1) if q < NQ - 1 else None

        def body(k, carry, q=q, mq=mq):
            b = 3 * k + 1
            block(q, b, (mq + 1) % 3, pf=(q, b + 2))
            block(q, b + 1, (mq + 2) % 3, pf=(q, b + 3))
            block(q, b + 2, mq, pf=(q, b + 4))
            return carry

        lax.fori_loop(0, (NBQ - 4) // 3, body, 0)
        block(q, NBQ - 3, (mq + NBQ - 3) % 3, pf=(q, NBQ - 1))
        if stage is not None:
            stage[0].wait()
            stage[1].wait()
            block(q, NBQ - 2, (mq + NBQ - 2) % 3, pf=(q + 1, 0))
            block(q, NBQ - 1, (mq + NBQ - 1) % 3, pf=(q + 1, 1))
        else:
            block(q, NBQ - 2, (mq + NBQ - 2) % 3)
            block(q, NBQ - 1, (mq + NBQ - 1) % 3)
    wait_scatter(NQ - 1, NBQ - 1, (NQ * NBQ - 1) % 3)

    plsc.subcore_barrier()
    @pl.when(s < NS - 1)
    def _():
        pltpu.sync_copy(acc.at[pl.ds(s * WR, WR)],
                        out_hbm.at[c, pl.ds(s * WR, WR)])

    @pl.when(s == NS - 1)
    def _():
        pltpu.sync_copy(acc.at[pl.ds((NS - 1) * WR, WRL)],
                        out_hbm.at[c, pl.ds((NS - 1) * WR, WRL)])


def _make_agg():
    mesh = plsc.VectorSubcoreMesh(core_axis_name="c", subcore_axis_name="s",
                                  num_cores=NC, num_subcores=NS)
    return functools.partial(
        pl.kernel,
        out_type=jax.ShapeDtypeStruct((NC, N, D), jnp.float32),
        mesh=mesh,
        scratch_types=[
            pltpu.VMEM((NBQ, BLK), jnp.int32),
            pltpu.VMEM((NBQ, BLK), jnp.int32),
            pltpu.VMEM((NBQ, BLK), jnp.int32),
            pltpu.VMEM((NBQ, BLK), jnp.int32),
            pltpu.VMEM((BLK, D), jnp.float32),
            pltpu.VMEM((BLK, D), jnp.float32),
            pltpu.VMEM((BLK, D), jnp.float32),
            pltpu.VMEM_SHARED((N, D), jnp.float32),
            pltpu.SemaphoreType.DMA,
            pltpu.SemaphoreType.DMA,
            pltpu.SemaphoreType.DMA,
            pltpu.SemaphoreType.DMA,
            pltpu.SemaphoreType.DMA,
            pltpu.SemaphoreType.DMA,
            pltpu.SemaphoreType.DMA,
        ],
    )(_agg_body)


_agg_cache = []


def _get_agg():
    if not _agg_cache:
        _agg_cache.append(_make_agg())
    return _agg_cache[0]


def _mlp1_body(agg_ref, x_ref, wa_ref, ba_ref, wb_ref, bb_ref, o_ref):
    h = agg_ref[0] + agg_ref[1] + x_ref[...]
    t = jnp.dot(h, wa_ref[...], preferred_element_type=jnp.float32) + ba_ref[...]
    t = jnp.maximum(t, 0.0)
    u = jnp.dot(t, wb_ref[...], preferred_element_type=jnp.float32) + bb_ref[...]
    o_ref[...] = jnp.maximum(u, 0.0)


def _mlp2_body(agg_ref, h_ref, wa_ref, ba_ref, wb_ref, bb_ref,
               wf_ref, bf_ref, o_ref):
    h = agg_ref[0] + agg_ref[1] + h_ref[...]
    t = jnp.dot(h, wa_ref[...], preferred_element_type=jnp.float32) + ba_ref[...]
    t = jnp.maximum(t, 0.0)
    u = jnp.dot(t, wb_ref[...], preferred_element_type=jnp.float32) + bb_ref[...]
    o_ref[...] = jnp.dot(u, wf_ref[...], preferred_element_type=jnp.float32) + bf_ref[...]


BN = 2000


def _w_spec():
    return pl.BlockSpec((D, D), lambda i: (0, 0))


def _b_spec():
    return pl.BlockSpec((D,), lambda i: (0,))


def _make_mlp1():
    return pl.pallas_call(
        _mlp1_body,
        grid=(N // BN,),
        in_specs=[
            pl.BlockSpec((NC, BN, D), lambda i: (0, i, 0)),
            pl.BlockSpec((BN, D), lambda i: (i, 0)),
            _w_spec(), _b_spec(), _w_spec(), _b_spec(),
        ],
        out_specs=pl.BlockSpec((BN, D), lambda i: (i, 0)),
        out_shape=jax.ShapeDtypeStruct((N, D), jnp.float32),
    )


def _make_mlp2():
    return pl.pallas_call(
        _mlp2_body,
        grid=(N // BN,),
        in_specs=[
            pl.BlockSpec((NC, BN, D), lambda i: (0, i, 0)),
            pl.BlockSpec((BN, D), lambda i: (i, 0)),
            _w_spec(), _b_spec(), _w_spec(), _b_spec(), _w_spec(), _b_spec(),
        ],
        out_specs=pl.BlockSpec((BN, D), lambda i: (i, 0)),
        out_shape=jax.ShapeDtypeStruct((N, D), jnp.float32),
    )


_mlp1 = _make_mlp1()
_mlp2 = _make_mlp2()


@jax.jit
def kernel(x, edge_index, W1a, b1a, W1b, b1b, W2a, b2a, W2b, b2b, Wfc, bfc):
    ei = edge_index.astype(jnp.int32).reshape(2, NW, NQ, NBQ, BLK)

    agg = _get_agg()
    agg1 = agg(x, ei)
    h1 = _mlp1(agg1, x, W1a, b1a, W1b, b1b)
    agg2 = agg(h1, ei)
    out = _mlp2(agg2, h1, W2a, b2a, W2b, b2b, Wfc, bfc)
    return out
